# Initial kernel scaffold; baseline (speedup 1.0000x reference)
import jax
import jax.numpy as jnp
from jax.experimental import pallas as pl


def kernel(x, node_ids, edge_index, batch, emb, Wp, bp, W0, b0, G0, B0, R0, rb0, W1, b1, G1, B1, R1, rb1, W2, b2, G2, B2, R2, rb2, Wout, bout, ln_g, ln_b):
    raise NotImplementedError("write your pallas kernel here")


# speedup vs baseline: 10.9490x; 10.9490x over previous
import functools
import math

import jax
import jax.numpy as jnp
from jax import lax
from jax.experimental import pallas as pl
from jax.experimental.pallas import tpu as pltpu
from jax.experimental.pallas import tpu_sc as plsc

N = 10000
E = 640000
D_IN = 128
H = 128
OUT = 768
VOCAB = 1000
G = 16
EPS = 1e-5

NC = 2
NS = 16
NW = NC * NS
CH = 128

E_PAD = 643072
EPW = E_PAD // NW
NCHUNK = EPW // CH

NID_PAD = 12288
ROWS_PER_W = NID_PAD // NW

ACC_ROWS = 10240
RPT = ACC_ROWS // NS
DUMMY = N

def _sc_mesh():
    return plsc.VectorSubcoreMesh(
        core_axis_name="c", subcore_axis_name="s",
        num_cores=NC, num_subcores=NS)


@functools.cache
def _get_sc_emb():
    return functools.partial(
        pl.kernel,
        out_type=jax.ShapeDtypeStruct((NID_PAD, H), jnp.float32),
        mesh=_sc_mesh(),
        scratch_types=[
            pltpu.VMEM((CH,), jnp.int32),
            pltpu.VMEM((CH, H), jnp.float32),
            pltpu.SemaphoreType.DMA,
        ],
    )(_sc_emb_body)


def _sc_emb_body(ids_hbm, emb_hbm, embrows_hbm, idx_v, rows_v, sem):
    ci = lax.axis_index("c")
    si = lax.axis_index("s")
    w = si * NC + ci
    gbase = w * ROWS_PER_W
    for k in range(ROWS_PER_W // CH):
        b = gbase + k * CH
        pltpu.sync_copy(ids_hbm.at[pl.ds(b, CH)], idx_v)
        pltpu.async_copy(emb_hbm.at[idx_v], rows_v, sem).wait()
        pltpu.sync_copy(rows_v, embrows_hbm.at[pl.ds(b, CH)])


@functools.cache
def _get_sc_deg():
    return functools.partial(
        pl.kernel,
        out_type=jax.ShapeDtypeStruct((NC, ACC_ROWS, H), jnp.float32),
        mesh=_sc_mesh(),
        scratch_types=[
            pltpu.VMEM((CH,), jnp.int32),
            pltpu.VMEM((CH, H), jnp.float32),
            pltpu.VMEM((CH, H), jnp.float32),
            pltpu.VMEM_SHARED((ACC_ROWS, H), jnp.float32),
        ],
    )(_sc_deg_body)


def _sc_deg_body(dst_hbm, oh_hbm, z_hbm, deg_hbm, idx_v, oh_v, z_v, acc_s):
    ci = lax.axis_index("c")
    si = lax.axis_index("s")
    w = si * NC + ci

    pltpu.sync_copy(oh_hbm, oh_v)
    pltpu.sync_copy(z_hbm, z_v)
    row0 = si * RPT
    for k in range(RPT // CH):
        pltpu.sync_copy(z_v, acc_s.at[pl.ds(row0 + k * CH, CH)])

    plsc.subcore_barrier()

    ebase = w * EPW

    def body(c, carry):
        pltpu.sync_copy(dst_hbm.at[pl.ds(ebase + c * CH, CH)], idx_v)
        pltpu.sync_copy(oh_v, acc_s.at[idx_v], add=True)
        return carry

    lax.fori_loop(0, NCHUNK, body, 0)

    plsc.subcore_barrier()
    for k in range(RPT // CH):
        r = row0 + k * CH
        pltpu.sync_copy(acc_s.at[pl.ds(r, CH)], deg_hbm.at[ci, pl.ds(r, CH)])


@functools.cache
def _get_sc_spmm():
    return functools.partial(
        pl.kernel,
        out_type=jax.ShapeDtypeStruct((NC, ACC_ROWS, H), jnp.float32),
        mesh=_sc_mesh(),
        scratch_types=[
            pltpu.VMEM((CH,), jnp.int32),
            pltpu.VMEM((CH,), jnp.int32),
            pltpu.VMEM((CH, H), jnp.float32),
            pltpu.VMEM((CH, H), jnp.float32),
            pltpu.VMEM_SHARED((ACC_ROWS, H), jnp.float32),
            pltpu.SemaphoreType.DMA,
        ],
    )(_sc_spmm_body)


def _sc_spmm_body(src_hbm, dst_hbm, z_hbm, table_hbm, parts_hbm,
                  sidx_v, didx_v, rows_v, z_v, acc_s, sem):
    ci = lax.axis_index("c")
    si = lax.axis_index("s")
    w = si * NC + ci

    pltpu.sync_copy(z_hbm, z_v)
    row0 = si * RPT
    for k in range(RPT // CH):
        pltpu.sync_copy(z_v, acc_s.at[pl.ds(row0 + k * CH, CH)])

    plsc.subcore_barrier()

    ebase = w * EPW

    def body(c, carry):
        base = ebase + c * CH
        pltpu.sync_copy(src_hbm.at[pl.ds(base, CH)], sidx_v)
        pltpu.sync_copy(dst_hbm.at[pl.ds(base, CH)], didx_v)
        pltpu.async_copy(table_hbm.at[sidx_v], rows_v, sem).wait()
        pltpu.sync_copy(rows_v, acc_s.at[didx_v], add=True)
        return carry

    lax.fori_loop(0, NCHUNK, body, 0)

    plsc.subcore_barrier()
    for k in range(RPT // CH):
        r = row0 + k * CH
        pltpu.sync_copy(acc_s.at[pl.ds(r, CH)], parts_hbm.at[ci, pl.ds(r, CH)])


_BLK = 1000


def _dot(a, b):
    return lax.dot_general(a, b, (((1,), (0,)), ((), ())),
                           precision=lax.Precision.HIGHEST,
                           preferred_element_type=jnp.float32)


def _dis_of(d_r):
    return lax.rsqrt(1.0 + d_r[...])


def _tc_init_body(x_r, er_r, d_r, wp_r, bp_r, w0_r, h_r, hwp_r):
    dis = _dis_of(d_r)
    h = _dot(x_r[...], wp_r[...]) + bp_r[...] + er_r[...]
    h_r[...] = h
    hwp_r[...] = _dot(h, w0_r[...]) * dis


def _tc_layer_body(h_r, hwp_r, p0_r, p1_r, d_r, r_r, rb_r, sc_r, sh_r,
                   wn_r, h2_r, hwp2_r):
    dis = _dis_of(d_r)
    s = p0_r[...] + p1_r[...] + hwp_r[...]
    z = jnp.maximum(dis * s * sc_r[...] + sh_r[...], 0.0)
    h2 = _dot(h_r[...], r_r[...]) + rb_r[...] + z
    h2_r[...] = h2
    hwp2_r[...] = _dot(h2, wn_r[...]) * dis


def _tc_last_body(h_r, hwp_r, p0_r, p1_r, d_r, r_r, rb_r, sc_r, sh_r,
                  h2_r):
    dis = _dis_of(d_r)
    s = p0_r[...] + p1_r[...] + hwp_r[...]
    z = jnp.maximum(dis * s * sc_r[...] + sh_r[...], 0.0)
    h2_r[...] = _dot(h_r[...], r_r[...]) + rb_r[...] + z


def _tc_pool_body(h_r, b_r, wout_r, bout_r, lng_r, lnb_r, out_r,
                  sums, maxs, cnts):
    i = pl.program_id(0)

    @pl.when(i == 0)
    def _init():
        sums[...] = jnp.zeros_like(sums)
        cnts[...] = jnp.zeros_like(cnts)
        maxs[...] = jnp.full_like(maxs, -jnp.inf)

    b = b_r[...]
    h = h_r[...]
    oh = (b == lax.broadcasted_iota(jnp.int32, (1, G), 1)).astype(jnp.float32)
    contract = (((0,), (0,)), ((), ()))
    sums[...] += lax.dot_general(oh, h, contract,
                                 precision=lax.Precision.HIGHEST,
                                 preferred_element_type=jnp.float32)
    cnts[...] += lax.dot_general(oh, jnp.ones_like(h), contract,
                                 precision=lax.Precision.HIGHEST,
                                 preferred_element_type=jnp.float32)
    blockmax = jnp.concatenate(
        [jnp.max(jnp.where(b == g, h, -jnp.inf), axis=0, keepdims=True)
         for g in range(G)], axis=0)
    maxs[...] = jnp.maximum(maxs[...], blockmax)

    @pl.when(i == pl.num_programs(0) - 1)
    def _fin():
        mean = sums[...] / jnp.maximum(cnts[...], 1.0)
        ge = jnp.concatenate([mean, maxs[...]], axis=1)
        y = _dot(ge, wout_r[...]) + bout_r[...]
        y = jnp.maximum(y, 0.0)
        mu = jnp.mean(y, axis=1, keepdims=True)
        var = jnp.mean((y - mu) ** 2, axis=1, keepdims=True)
        out_r[...] = (y - mu) * lax.rsqrt(var + EPS) * lng_r[...] + lnb_r[...]


def _row_spec(cols):
    return pl.BlockSpec((_BLK, cols), lambda i: (i, 0))


def _full_spec(rows, cols):
    return pl.BlockSpec((rows, cols), lambda i: (0, 0))


def _tc_init(x, embrows, dcol, Wp, bp, W0):
    return pl.pallas_call(
        _tc_init_body,
        grid=(N // _BLK,),
        in_specs=[_row_spec(H), _row_spec(H), _row_spec(1),
                  _full_spec(D_IN, H), _full_spec(1, H), _full_spec(H, H)],
        out_specs=[_row_spec(H), _row_spec(H)],
        out_shape=[jax.ShapeDtypeStruct((N, H), jnp.float32)] * 2,
    )(x, embrows, dcol, Wp, bp, W0)


def _tc_layer(h, hwp, p0, p1, dcol, R, rb, scale, shift, Wn):
    return pl.pallas_call(
        _tc_layer_body,
        grid=(N // _BLK,),
        in_specs=[_row_spec(H)] * 4 + [_row_spec(1)] +
                 [_full_spec(H, H), _full_spec(1, H), _full_spec(1, H),
                  _full_spec(1, H), _full_spec(H, H)],
        out_specs=[_row_spec(H), _row_spec(H)],
        out_shape=[jax.ShapeDtypeStruct((N, H), jnp.float32)] * 2,
    )(h, hwp, p0, p1, dcol, R, rb, scale, shift, Wn)


def _tc_last(h, hwp, p0, p1, dcol, R, rb, scale, shift):
    return pl.pallas_call(
        _tc_last_body,
        grid=(N // _BLK,),
        in_specs=[_row_spec(H)] * 4 + [_row_spec(1)] +
                 [_full_spec(H, H), _full_spec(1, H), _full_spec(1, H),
                  _full_spec(1, H)],
        out_specs=[_row_spec(H)],
        out_shape=[jax.ShapeDtypeStruct((N, H), jnp.float32)],
    )(h, hwp, p0, p1, dcol, R, rb, scale, shift)[0]


def _tc_pool(h, batch2d, Wout, bout, ln_g, ln_b):
    return pl.pallas_call(
        _tc_pool_body,
        grid=(N // _BLK,),
        in_specs=[_row_spec(H), _row_spec(1),
                  _full_spec(2 * H, OUT), _full_spec(1, OUT),
                  _full_spec(1, OUT), _full_spec(1, OUT)],
        out_specs=[_full_spec(G, OUT)],
        out_shape=[jax.ShapeDtypeStruct((G, OUT), jnp.float32)],
        scratch_shapes=[pltpu.VMEM((G, H), jnp.float32),
                        pltpu.VMEM((G, H), jnp.float32),
                        pltpu.VMEM((G, H), jnp.float32)],
    )(h, batch2d, Wout, bout, ln_g, ln_b)[0]


def kernel(x, node_ids, edge_index, batch, emb, Wp, bp,
           W0, b0, G0, B0, R0, rb0,
           W1, b1, G1, B1, R1, rb1,
           W2, b2, G2, B2, R2, rb2,
           Wout, bout, ln_g, ln_b):
    f32 = jnp.float32
    src = edge_index[0].astype(jnp.int32)
    dst = edge_index[1].astype(jnp.int32)
    src_p = jnp.concatenate([src, jnp.zeros((E_PAD - E,), jnp.int32)])
    dst_p = jnp.concatenate([dst, jnp.full((E_PAD - E,), DUMMY, jnp.int32)])
    ids_p = jnp.concatenate(
        [node_ids.astype(jnp.int32), jnp.zeros((NID_PAD - N,), jnp.int32)])

    ohH = jnp.concatenate(
        [jnp.ones((CH, 1), f32), jnp.zeros((CH, H - 1), f32)], axis=1)
    zH = jnp.zeros((CH, H), f32)

    embrows = _get_sc_emb()(ids_p, emb)[:N]
    degp = _get_sc_deg()(dst_p, ohH, zH)
    dcol = degp[0, :N, 0:1] + degp[1, :N, 0:1]

    cbn = 1.0 / math.sqrt(1.0 + EPS)
    bp2 = bp.reshape(1, H).astype(f32)
    scales = [(cbn * g).reshape(1, H) for g in (G0, G1, G2)]
    shifts = [(b * cbn * g + bb).reshape(1, H)
              for (b, g, bb) in ((b0, G0, B0), (b1, G1, B1), (b2, G2, B2))]

    h, hwp = _tc_init(x, embrows, dcol, Wp, bp2, W0)

    parts = _get_sc_spmm()(src_p, dst_p, zH, hwp)
    h, hwp = _tc_layer(h, hwp, parts[0, :N], parts[1, :N], dcol,
                       R0, rb0.reshape(1, H), scales[0], shifts[0], W1)

    parts = _get_sc_spmm()(src_p, dst_p, zH, hwp)
    h, hwp = _tc_layer(h, hwp, parts[0, :N], parts[1, :N], dcol,
                       R1, rb1.reshape(1, H), scales[1], shifts[1], W2)

    parts = _get_sc_spmm()(src_p, dst_p, zH, hwp)
    h = _tc_last(h, hwp, parts[0, :N], parts[1, :N], dcol,
                 R2, rb2.reshape(1, H), scales[2], shifts[2])

    batch2d = batch.astype(jnp.int32).reshape(N, 1)
    return _tc_pool(h, batch2d, Wout, bout.reshape(1, OUT),
                    ln_g.reshape(1, OUT), ln_b.reshape(1, OUT))
